# Initial kernel scaffold; baseline (speedup 1.0000x reference)
import jax
import jax.numpy as jnp
from jax.experimental import pallas as pl


def kernel(logits, illegal_action_masks, conflict_pairs):
    raise NotImplementedError("write your pallas kernel here")


# speedup vs baseline: 1.0169x; 1.0169x over previous
import functools

import jax
import jax.numpy as jnp
from jax.experimental import pallas as pl
from jax.experimental.pallas import tpu as pltpu

ITERS = 5
COEF = -100.0
BIG = 10000000000.0

ROW_BLOCK = 1000


def _softmax_body(base_ref, pun_ref, probs_ref):
    q = base_ref[...] + COEF * pun_ref[...]
    m = jnp.max(q, axis=-1, keepdims=True)
    e = jnp.exp(q - m)
    probs_ref[...] = e / jnp.sum(e, axis=-1, keepdims=True)


def _softmax(base, pun):
    N, A = base.shape
    grid = (N // ROW_BLOCK,)
    return pl.pallas_call(
        _softmax_body,
        grid=grid,
        in_specs=[
            pl.BlockSpec((ROW_BLOCK, A), lambda i: (i, 0)),
            pl.BlockSpec((ROW_BLOCK, A), lambda i: (i, 0)),
        ],
        out_specs=pl.BlockSpec((ROW_BLOCK, A), lambda i: (i, 0)),
        out_shape=jax.ShapeDtypeStruct((N, A), base.dtype),
    )(base, pun)


def kernel(logits, illegal_action_masks, conflict_pairs):
    N, A = logits.shape
    cp = conflict_pairs.astype(jnp.int32)
    index1 = cp[:, 0] * A + cp[:, 1]
    index2 = cp[:, 2] * A + cp[:, 3]
    masks = cp[:, 4].astype(logits.dtype)

    mask_pen = -BIG * illegal_action_masks
    ml = logits + mask_pen

    base = mask_pen
    pun = jnp.zeros((N, A), dtype=logits.dtype)
    for it in range(ITERS):
        probs = _softmax(base, pun)
        gathered = jnp.take(probs.reshape(-1), index2)
        pun = (
            jnp.zeros((N * A,), dtype=logits.dtype)
            .at[index1]
            .add(gathered * masks)
            .reshape(N, A)
        )
        base = ml
    return ml + COEF * pun


# speedup vs baseline: 4.7593x; 4.6804x over previous
import functools

import jax
import jax.numpy as jnp
from jax import lax
from jax.experimental import pallas as pl
from jax.experimental.pallas import tpu as pltpu
from jax.experimental.pallas import tpu_sc as plsc

ITERS = 5
COEF = -100.0
BIG = 10000000000.0

ROW_BLOCK = 1000

NC, NS = 2, 16
NW = NC * NS
CHUNK = 2048
CPT = 25
TILE_PAIRS = CPT * CHUNK
PAD_P = NW * TILE_PAIRS
NRANGE = 8
RANGE = 1600000
RS = RANGE // NS
ZB = 2000
GSUB = 1024


def _softmax_body(base_ref, pun_ref, probs_ref):
    q = base_ref[...] + COEF * pun_ref[...]
    m = jnp.max(q, axis=-1, keepdims=True)
    e = jnp.exp(q - m)
    probs_ref[...] = e / jnp.sum(e, axis=-1, keepdims=True)


def _softmax(base, pun):
    N, A = base.shape
    grid = (N // ROW_BLOCK,)
    return pl.pallas_call(
        _softmax_body,
        grid=grid,
        in_specs=[
            pl.BlockSpec((ROW_BLOCK, A), lambda i: (i, 0)),
            pl.BlockSpec((ROW_BLOCK, A), lambda i: (i, 0)),
        ],
        out_specs=pl.BlockSpec((ROW_BLOCK, A), lambda i: (i, 0)),
        out_shape=jax.ShapeDtypeStruct((N, A), base.dtype),
    )(base, pun)


def _final_body(base_ref, pun_ref, out_ref):
    out_ref[...] = base_ref[...] + COEF * pun_ref[...]


def _final(base, pun):
    N, A = base.shape
    grid = (N // ROW_BLOCK,)
    return pl.pallas_call(
        _final_body,
        grid=grid,
        in_specs=[
            pl.BlockSpec((ROW_BLOCK, A), lambda i: (i, 0)),
            pl.BlockSpec((ROW_BLOCK, A), lambda i: (i, 0)),
        ],
        out_specs=pl.BlockSpec((ROW_BLOCK, A), lambda i: (i, 0)),
        out_shape=jax.ShapeDtypeStruct((N, A), base.dtype),
    )(base, pun)


def _gather_body(probs_hbm, idx2_hbm, mask_hbm, vals_hbm,
                 buf_i, buf_f, val_g, sem):
    c = lax.axis_index("c")
    s = lax.axis_index("s")
    wid = s * NC + c

    def gchunk(ch, _):
        blk = wid * CPT + ch
        pltpu.sync_copy(idx2_hbm.at[blk], buf_i)
        cp = pltpu.async_copy(probs_hbm.at[buf_i], val_g, sem)
        pltpu.sync_copy(mask_hbm.at[blk], buf_f)
        cp.wait()
        for g in range(CHUNK // 16):
            sl = pl.ds(g * 16, 16)
            buf_f[sl] = val_g[sl] * buf_f[sl]
        pltpu.sync_copy(buf_f, vals_hbm.at[blk])
        return _
    lax.fori_loop(0, CPT, gchunk, 0)


@functools.partial(
    pl.kernel,
    out_type=jax.ShapeDtypeStruct((NW * CPT, CHUNK), jnp.float32),
    mesh=plsc.VectorSubcoreMesh(core_axis_name="c", subcore_axis_name="s"),
    scratch_types=[
        pltpu.VMEM((CHUNK,), jnp.int32),
        pltpu.VMEM((CHUNK,), jnp.float32),
        pltpu.VMEM((CHUNK,), jnp.float32),
        pltpu.SemaphoreType.DMA,
    ],
)
def _sc_gather(probs_hbm, idx2_hbm, mask_hbm, vals_hbm,
               buf_i, buf_f, val_g, sem):
    _gather_body(probs_hbm, idx2_hbm, mask_hbm, vals_hbm,
                 buf_i, buf_f, val_g, sem)


def _scatter_body(idx1_hbm, vals_hbm, out_hbm, buf_i, buf_f, val_g,
                  acc_sh, sem):
    c = lax.axis_index("c")
    s = lax.axis_index("s")
    lanes = jax.lax.iota(jnp.int32, 16)
    cpp = (NW * CPT) // NS

    for p in range(NRANGE // NC):
        rng = c * (NRANGE // NC) + p
        base = rng * RANGE

        for g in range(CHUNK // 16):
            val_g[pl.ds(g * 16, 16)] = jnp.zeros((16,), jnp.float32)

        def zloop(z, _):
            pltpu.sync_copy(val_g.at[pl.ds(0, ZB)],
                            acc_sh.at[pl.ds(s * RS + z * ZB, ZB)])
            return _
        lax.fori_loop(0, RS // ZB, zloop, 0)
        plsc.subcore_barrier()

        def schunk(ch, _):
            blk = s * cpp + ch
            pltpu.sync_copy(idx1_hbm.at[blk], buf_i)
            pltpu.sync_copy(vals_hbm.at[blk], buf_f)
            for g in range(CHUNK // 16):
                sl = pl.ds(g * 16, 16)
                iv = buf_i[sl]
                rel = iv - base
                inb = (rel >= 0) & (rel < RANGE)
                buf_i[sl] = jnp.where(inb, rel, RANGE + s * 16 + lanes)
                buf_f[sl] = jnp.where(
                    inb, buf_f[sl], jnp.zeros((16,), jnp.float32))
            pltpu.sync_copy(buf_f, acc_sh.at[buf_i], add=True)
            return _
        lax.fori_loop(0, cpp, schunk, 0)
        plsc.subcore_barrier()

        def wloop(z, _):
            pltpu.sync_copy(acc_sh.at[pl.ds(s * RS + z * ZB, ZB)],
                            val_g.at[pl.ds(0, ZB)])
            pltpu.sync_copy(val_g.at[pl.ds(0, ZB)],
                            out_hbm.at[pl.ds(base + s * RS + z * ZB, ZB)])
            return _
        lax.fori_loop(0, RS // ZB, wloop, 0)
        plsc.subcore_barrier()


@functools.partial(
    pl.kernel,
    out_type=jax.ShapeDtypeStruct((NRANGE * RANGE,), jnp.float32),
    mesh=plsc.VectorSubcoreMesh(core_axis_name="c", subcore_axis_name="s"),
    scratch_types=[
        pltpu.VMEM((CHUNK,), jnp.int32),
        pltpu.VMEM((CHUNK,), jnp.float32),
        pltpu.VMEM((CHUNK,), jnp.float32),
        pltpu.VMEM_SHARED((RANGE + NS * 16,), jnp.float32),
        pltpu.SemaphoreType.DMA,
    ],
)
def _sc_scatter(idx1_hbm, vals_hbm, out_hbm, buf_i, buf_f, val_g,
                acc_sh, sem):
    _scatter_body(idx1_hbm, vals_hbm, out_hbm, buf_i, buf_f, val_g,
                  acc_sh, sem)


def kernel(logits, illegal_action_masks, conflict_pairs):
    N, A = logits.shape
    cp = conflict_pairs.astype(jnp.int32)
    index1 = cp[:, 0] * A + cp[:, 1]
    index2 = cp[:, 2] * A + cp[:, 3]
    masks = cp[:, 4].astype(logits.dtype)

    P = index1.shape[0]
    pad = PAD_P - P
    idx1p = jnp.pad(index1, (0, pad)).reshape(NW * CPT, CHUNK)
    idx2p = jnp.pad(index2, (0, pad)).reshape(NW * CPT, CHUNK)
    maskp = jnp.pad(masks, (0, pad)).reshape(NW * CPT, CHUNK)

    mask_pen = -BIG * illegal_action_masks
    ml = logits + mask_pen

    base = mask_pen
    pun = jnp.zeros((N, A), dtype=logits.dtype)
    for it in range(ITERS):
        probs = _softmax(base, pun)
        vals = _sc_gather(probs.reshape(-1), idx2p, maskp)
        pun = _sc_scatter(idx1p, vals).reshape(N, A)
        base = ml
    return _final(ml, pun)


# speedup vs baseline: 5.6429x; 1.1857x over previous
import functools

import jax
import jax.numpy as jnp
from jax import lax
from jax.experimental import pallas as pl
from jax.experimental.pallas import tpu as pltpu
from jax.experimental.pallas import tpu_sc as plsc

ITERS = 5
COEF = -100.0
BIG = 10000000000.0

ROW_BLOCK = 1000

NC, NS = 2, 16
NW = NC * NS
CHUNK = 2048
CPT = 25
TILE_PAIRS = CPT * CHUNK
PAD_P = NW * TILE_PAIRS
NRANGE = 8
RANGE = 1600000
RS = RANGE // NS
ZB = 2000
GSUB = 1024


def _softmax_body(base_ref, pun_ref, probs_ref):
    q = base_ref[...] + COEF * pun_ref[...]
    m = jnp.max(q, axis=-1, keepdims=True)
    e = jnp.exp(q - m)
    probs_ref[...] = e / jnp.sum(e, axis=-1, keepdims=True)


def _softmax(base, pun):
    N, A = base.shape
    grid = (N // ROW_BLOCK,)
    return pl.pallas_call(
        _softmax_body,
        grid=grid,
        in_specs=[
            pl.BlockSpec((ROW_BLOCK, A), lambda i: (i, 0)),
            pl.BlockSpec((ROW_BLOCK, A), lambda i: (i, 0)),
        ],
        out_specs=pl.BlockSpec((ROW_BLOCK, A), lambda i: (i, 0)),
        out_shape=jax.ShapeDtypeStruct((N, A), base.dtype),
    )(base, pun)


def _final_body(base_ref, pun_ref, out_ref):
    out_ref[...] = base_ref[...] + COEF * pun_ref[...]


def _final(base, pun):
    N, A = base.shape
    grid = (N // ROW_BLOCK,)
    return pl.pallas_call(
        _final_body,
        grid=grid,
        in_specs=[
            pl.BlockSpec((ROW_BLOCK, A), lambda i: (i, 0)),
            pl.BlockSpec((ROW_BLOCK, A), lambda i: (i, 0)),
        ],
        out_specs=pl.BlockSpec((ROW_BLOCK, A), lambda i: (i, 0)),
        out_shape=jax.ShapeDtypeStruct((N, A), base.dtype),
    )(base, pun)


def _gather_body(probs_hbm, idx2_hbm, mask_hbm, vals_hbm,
                 buf_i, buf_f, val_g, sem):
    c = lax.axis_index("c")
    s = lax.axis_index("s")
    wid = s * NC + c

    def gchunk(ch, _):
        blk = wid * CPT + ch
        pltpu.sync_copy(idx2_hbm.at[blk], buf_i)
        cp = pltpu.async_copy(probs_hbm.at[buf_i], val_g, sem)
        pltpu.sync_copy(mask_hbm.at[blk], buf_f)
        cp.wait()
        for g in range(CHUNK // 16):
            sl = pl.ds(g * 16, 16)
            buf_f[sl] = val_g[sl] * buf_f[sl]
        pltpu.sync_copy(buf_f, vals_hbm.at[blk])
        return _
    lax.fori_loop(0, CPT, gchunk, 0)


@functools.partial(
    pl.kernel,
    out_type=jax.ShapeDtypeStruct((NW * CPT, CHUNK), jnp.float32),
    mesh=plsc.VectorSubcoreMesh(core_axis_name="c", subcore_axis_name="s"),
    scratch_types=[
        pltpu.VMEM((CHUNK,), jnp.int32),
        pltpu.VMEM((CHUNK,), jnp.float32),
        pltpu.VMEM((CHUNK,), jnp.float32),
        pltpu.SemaphoreType.DMA,
    ],
)
def _sc_gather(probs_hbm, idx2_hbm, mask_hbm, vals_hbm,
               buf_i, buf_f, val_g, sem):
    _gather_body(probs_hbm, idx2_hbm, mask_hbm, vals_hbm,
                 buf_i, buf_f, val_g, sem)


def _scatter_body(idx1_hbm, vals_hbm, out_hbm, buf_i, buf_f, val_g,
                  acc_sh, sem):
    c = lax.axis_index("c")
    s = lax.axis_index("s")
    lanes = jax.lax.iota(jnp.int32, 16)
    cpp = (NW * CPT) // NS

    for p in range(NRANGE // NC):
        rng = c * (NRANGE // NC) + p
        base = rng * RANGE

        for g in range(CHUNK // 16):
            val_g[pl.ds(g * 16, 16)] = jnp.zeros((16,), jnp.float32)

        def zloop(z, _):
            pltpu.sync_copy(val_g.at[pl.ds(0, ZB)],
                            acc_sh.at[pl.ds(s * RS + z * ZB, ZB)])
            return _
        lax.fori_loop(0, RS // ZB, zloop, 0)
        plsc.subcore_barrier()

        def schunk(ch, _):
            blk = s * cpp + ch
            pltpu.sync_copy(idx1_hbm.at[blk], buf_i)
            pltpu.sync_copy(vals_hbm.at[blk], buf_f)
            for g in range(CHUNK // 16):
                sl = pl.ds(g * 16, 16)
                iv = buf_i[sl]
                rel = iv - base
                inb = (rel >= 0) & (rel < RANGE)
                buf_i[sl] = jnp.where(inb, rel, RANGE + s * 16 + lanes)
                buf_f[sl] = jnp.where(
                    inb, buf_f[sl], jnp.zeros((16,), jnp.float32))
            pltpu.sync_copy(buf_f, acc_sh.at[buf_i], add=True)
            return _
        lax.fori_loop(0, cpp, schunk, 0)
        plsc.subcore_barrier()

        def wloop(z, _):
            pltpu.sync_copy(acc_sh.at[pl.ds(s * RS + z * ZB, ZB)],
                            val_g.at[pl.ds(0, ZB)])
            pltpu.sync_copy(val_g.at[pl.ds(0, ZB)],
                            out_hbm.at[pl.ds(base + s * RS + z * ZB, ZB)])
            return _
        lax.fori_loop(0, RS // ZB, wloop, 0)
        plsc.subcore_barrier()


@functools.partial(
    pl.kernel,
    out_type=jax.ShapeDtypeStruct((NRANGE * RANGE,), jnp.float32),
    mesh=plsc.VectorSubcoreMesh(core_axis_name="c", subcore_axis_name="s"),
    scratch_types=[
        pltpu.VMEM((CHUNK,), jnp.int32),
        pltpu.VMEM((CHUNK,), jnp.float32),
        pltpu.VMEM((CHUNK,), jnp.float32),
        pltpu.VMEM_SHARED((RANGE + NS * 16,), jnp.float32),
        pltpu.SemaphoreType.DMA,
    ],
)
def _sc_scatter(idx1_hbm, vals_hbm, out_hbm, buf_i, buf_f, val_g,
                acc_sh, sem):
    _scatter_body(idx1_hbm, vals_hbm, out_hbm, buf_i, buf_f, val_g,
                  acc_sh, sem)


def kernel(logits, illegal_action_masks, conflict_pairs):
    N, A = logits.shape
    cp = conflict_pairs.astype(jnp.int32)
    index1 = cp[:, 0] * A + cp[:, 1]
    index2 = cp[:, 2] * A + cp[:, 3]
    masks = cp[:, 4].astype(logits.dtype)

    P = index1.shape[0]
    pad = PAD_P - P
    idx1p = jnp.pad(index1, (0, pad)).reshape(NW * CPT, CHUNK)
    idx2p = jnp.concatenate(
        [index2, jnp.arange(pad, dtype=jnp.int32)]).reshape(NW * CPT, CHUNK)
    maskp = jnp.pad(masks, (0, pad)).reshape(NW * CPT, CHUNK)

    mask_pen = -BIG * illegal_action_masks
    ml = logits + mask_pen

    base = mask_pen
    pun = jnp.zeros((N, A), dtype=logits.dtype)
    for it in range(ITERS):
        probs = _softmax(base, pun)
        vals = _sc_gather(probs.reshape(-1), idx2p, maskp)
        pun = _sc_scatter(idx1p, vals).reshape(N, A)
        base = ml
    return _final(ml, pun)
